# baseline (speedup 1.0000x reference)
import functools

import jax
import jax.numpy as jnp
from jax import lax
from jax.experimental import pallas as pl
from jax.experimental.pallas import tpu as pltpu
from jax.experimental.pallas import tpu_sc as plsc

F32 = jnp.float32
N_GRAPHS = 256
IDXBLK = 128
LANES = 128


def _pre_body(x_ref, scale_ref, beta_ref, root1_ref, bias1_ref,
              xq_ref, r1p_ref):
    n, f = x_ref.shape
    xn = x_ref[...] * scale_ref[...] + beta_ref[...]
    x4 = xn.reshape(n // 4, 4, f)
    xq_ref[...] = jnp.concatenate([x4[:, k, :] for k in range(4)], axis=1)
    x2 = xn.reshape(n // 2, 2, f)
    r1e = jnp.dot(x2[:, 0, :], root1_ref[...], preferred_element_type=F32) + bias1_ref[...]
    r1o = jnp.dot(x2[:, 1, :], root1_ref[...], preferred_element_type=F32) + bias1_ref[...]
    r1p_ref[...] = jnp.concatenate([r1e, r1o], axis=1)


def _msg1_body(ea0_ref, ea1_ref, ea2_ref, ea3_ref, xs_ref, s_ref, r_ref, v_ref,
               out_ref, *, f_in):
    eas = (ea0_ref, ea1_ref, ea2_ref, ea3_ref)
    ms = []
    for k in range(4):
        xk = xs_ref[:, k * f_in:(k + 1) * f_in]
        e7 = lax.dot_general(eas[k][...], s_ref[...], (((0,), (0,)), ((), ())),
                             preferred_element_type=F32)
        z = e7 * jnp.dot(xk, r_ref[...], preferred_element_type=F32)
        ms.append(jnp.dot(z, v_ref[...], preferred_element_type=F32))
    a = jnp.concatenate([ms[0], ms[1]], axis=1)
    b = jnp.concatenate([ms[2], ms[3]], axis=1)
    inter = jnp.concatenate([a[:, None, :], b[:, None, :]], axis=1)
    out_ref[...] = inter.reshape(out_ref.shape)


def _msg2_body(eae_ref, eao_ref, xs_ref, s_ref, r_ref, v_ref, out_ref, *, f_in):
    ms = []
    for k, ea in enumerate((eae_ref, eao_ref)):
        xk = xs_ref[:, k * f_in:(k + 1) * f_in]
        e7 = lax.dot_general(ea[...], s_ref[...], (((0,), (0,)), ((), ())),
                             preferred_element_type=F32)
        z = e7 * jnp.dot(xk, r_ref[...], preferred_element_type=F32)
        ms.append(jnp.dot(z, v_ref[...], preferred_element_type=F32))
    out_ref[...] = jnp.concatenate(ms, axis=1)


def _hidden_body(agg_ref, r1p_ref, root2_ref, bias2_ref, h1p_ref, r2p_ref):
    h = root2_ref.shape[0]
    he = jnp.maximum(agg_ref[0][:, :h] + agg_ref[1][:, :h] + r1p_ref[:, :h], 0.0)
    ho = jnp.maximum(agg_ref[0][:, h:] + agg_ref[1][:, h:] + r1p_ref[:, h:], 0.0)
    h1p_ref[...] = jnp.concatenate([he, ho], axis=1)
    r2e = jnp.dot(he, root2_ref[...], preferred_element_type=F32) + bias2_ref[...]
    r2o = jnp.dot(ho, root2_ref[...], preferred_element_type=F32) + bias2_ref[...]
    r2p_ref[...] = jnp.concatenate([r2e, r2o], axis=1)


def _pool_body(agg_ref, r2p_ref, wf_ref, bf_ref, wa_ref, ba_ref, wd_ref, bd_ref,
               sege_ref, sego_ref, out_ref, acc_ref, *, n_graphs, nblocks, h):
    j = pl.program_id(0)
    part = None
    for k, seg_ref in enumerate((sege_ref, sego_ref)):
        hk = jnp.maximum(agg_ref[0][:, k * h:(k + 1) * h]
                         + agg_ref[1][:, k * h:(k + 1) * h]
                         + r2p_ref[:, k * h:(k + 1) * h], 0.0)
        feat = jnp.dot(hk, wf_ref[...], preferred_element_type=F32) + bf_ref[...]
        attn = jax.nn.sigmoid(jnp.dot(hk, wa_ref[...], preferred_element_type=F32)
                              + ba_ref[...])
        p = feat * attn
        seg = seg_ref[...]
        onehot = (seg == lax.broadcasted_iota(
            jnp.int32, (n_graphs, seg.shape[1]), 0)).astype(F32)
        pk = jnp.dot(onehot, p, preferred_element_type=F32)
        part = pk if part is None else part + pk

    @pl.when(j == 0)
    def _():
        acc_ref[...] = part

    @pl.when(j > 0)
    def _():
        acc_ref[...] = acc_ref[...] + part

    @pl.when(j == nblocks - 1)
    def _():
        out_ref[...] = (jnp.dot(acc_ref[...], wd_ref[...], preferred_element_type=F32)
                        + bd_ref[...])


_SC_PARAMS = pltpu.CompilerParams(use_tc_tiling_on_sc=False)


def _sc_gather(table, idx2d, fd):
    nrows_idx, _ = idx2d.shape
    e_total = nrows_idx * IDXBLK
    info = plsc.get_sparse_core_info()
    nc, ns = info.num_cores, info.num_subcores
    nw = nc * ns
    chunk = e_total // nw
    kblk = chunk // IDXBLK
    nparts = 4
    kpart = kblk // nparts
    prows = chunk // nparts
    nbuf = 2
    mesh = plsc.VectorSubcoreMesh(core_axis_name="c", subcore_axis_name="s")

    @functools.partial(
        pl.kernel,
        out_type=jax.ShapeDtypeStruct((e_total, fd), F32),
        mesh=mesh,
        compiler_params=_SC_PARAMS,
        scratch_types=[
            pltpu.VMEM((kblk, IDXBLK), jnp.int32),
            [pltpu.VMEM((prows, fd), F32) for _ in range(nbuf)],
            [pltpu.SemaphoreType.DMA for _ in range(nbuf)],
            [pltpu.SemaphoreType.DMA for _ in range(nbuf)],
        ],
    )
    def gk(table_hbm, idx_hbm, out_hbm, idx_v, bufs, gsems, osems):
        c = lax.axis_index("c")
        s = lax.axis_index("s")
        w = s * nc + c
        pltpu.sync_copy(idx_hbm.at[pl.ds(w * kblk, kblk)], idx_v)
        outcp = [None] * nbuf
        for p in range(nparts):
            b = p % nbuf
            if outcp[b] is not None:
                outcp[b].wait()
            cps = []
            for j in range(kpart):
                cps.append(pltpu.async_copy(
                    table_hbm.at[idx_v.at[p * kpart + j]],
                    bufs[b].at[pl.ds(j * IDXBLK, IDXBLK)], gsems[b]))
            for cp in cps:
                cp.wait()
            outcp[b] = pltpu.async_copy(
                bufs[b], out_hbm.at[pl.ds(w * chunk + p * prows, prows)], osems[b])
        for cp in outcp:
            cp.wait()

    return gk(table, idx2d)


def _sc_scatter_add(vals, idx2d, n_nodes, fd):
    nrows_idx, _ = idx2d.shape
    e_total = nrows_idx * IDXBLK
    info = plsc.get_sparse_core_info()
    nc, ns = info.num_cores, info.num_subcores
    nw = nc * ns
    chunk = e_total // nw
    kblk = chunk // IDXBLK
    nparts = kblk
    rows_per_tile = n_nodes // ns
    mesh = plsc.VectorSubcoreMesh(core_axis_name="c", subcore_axis_name="s")

    zrows = 16
    nbuf = 3

    @functools.partial(
        pl.kernel,
        out_type=jax.ShapeDtypeStruct((nc, n_nodes, fd), F32),
        mesh=mesh,
        compiler_params=_SC_PARAMS,
        scratch_types=[
            pltpu.VMEM((kblk, IDXBLK), jnp.int32),
            [pltpu.VMEM((IDXBLK, fd), F32) for _ in range(nbuf)],
            pltpu.VMEM((zrows, fd), F32),
            pltpu.VMEM_SHARED((n_nodes, fd), F32),
            [pltpu.SemaphoreType.DMA for _ in range(nbuf)],
            [pltpu.SemaphoreType.DMA for _ in range(nbuf)],
        ],
    )
    def sk(vals_hbm, idx_hbm, out_hbm, idx_v, bufs, zbuf, acc_sh, lsems, asems):
        c = lax.axis_index("c")
        s = lax.axis_index("s")
        w = s * nc + c
        r0 = s * rows_per_tile
        pltpu.sync_copy(idx_hbm.at[pl.ds(w * kblk, kblk)], idx_v)
        loadcp = [None] * nparts
        for p in range(nbuf):
            loadcp[p] = pltpu.async_copy(
                vals_hbm.at[pl.ds(w * chunk + p * IDXBLK, IDXBLK)],
                bufs[p], lsems[p])
        nlane16 = fd // 16

        def bz(k, _):
            zbuf[k // nlane16, pl.ds((k % nlane16) * 16, 16)] = jnp.zeros((16,), F32)
            return 0

        lax.fori_loop(0, zrows * nlane16, bz, 0)
        for t in range(rows_per_tile // zrows):
            pltpu.sync_copy(zbuf, acc_sh.at[pl.ds(r0 + t * zrows, zrows)])
        plsc.subcore_barrier()
        addcp = [None] * nparts
        for p in range(nparts):
            b = p % nbuf
            loadcp[p].wait()
            addcp[p] = pltpu.async_copy(bufs[b], acc_sh.at[idx_v.at[p]],
                                        asems[b], add=True)
            q = p + nbuf
            if q < nparts:
                addcp[p].wait()
                loadcp[q] = pltpu.async_copy(
                    vals_hbm.at[pl.ds(w * chunk + q * IDXBLK, IDXBLK)],
                    bufs[b], lsems[b])
        for p in range(nparts - nbuf, nparts):
            addcp[p].wait()
        plsc.subcore_barrier()
        pltpu.sync_copy(acc_sh.at[pl.ds(r0, rows_per_tile)],
                        out_hbm.at[c, pl.ds(r0, rows_per_tile)])

    return sk(vals, idx2d)


def kernel(x, e, gamma, beta, W1e, b1e, root1, bias1, W2e, b2e, root2, bias2,
           Wf, bf, Wa, ba, Wd, bd, edge_index, i):
    n, f_in = x.shape
    e_total, d_edge = e.shape
    h = root1.shape[1]
    p_ch = Wf.shape[1]
    n_out = Wd.shape[1]

    scale = (gamma * lax.rsqrt(jnp.float32(1.0 + 1e-3))).reshape(1, f_in)
    beta2 = beta.reshape(1, f_in)
    src2d = edge_index[0].reshape(e_total // IDXBLK, IDXBLK)
    dst2d = edge_index[1].reshape(e_total // IDXBLK, IDXBLK)
    dd = d_edge + 1
    ea_t = jnp.concatenate([e.T, jnp.ones((1, e_total), F32)], axis=0)
    ea_q = [ea_t[:, k::4] for k in range(4)]
    ea_e = ea_t[:, 0::2]
    ea_o = ea_t[:, 1::2]
    v1 = jnp.concatenate([W1e, b1e[None, :]], axis=0).reshape(dd * f_in, h)
    v2 = jnp.concatenate([W2e, b2e[None, :]], axis=0).reshape(dd * h, h)
    s1 = jnp.kron(jnp.eye(dd, dtype=F32), jnp.ones((1, f_in), F32))
    r1m = jnp.tile(jnp.eye(f_in, dtype=F32), (1, dd))
    s2 = jnp.kron(jnp.eye(dd, dtype=F32), jnp.ones((1, h), F32))
    r2m = jnp.tile(jnp.eye(h, dtype=F32), (1, dd))
    seg_e = i[0::2].reshape(1, n // 2)
    seg_o = i[1::2].reshape(1, n // 2)

    xq, r1p = pl.pallas_call(
        _pre_body,
        out_shape=[jax.ShapeDtypeStruct((n // 4, 4 * f_in), F32),
                   jax.ShapeDtypeStruct((n // 2, 2 * h), F32)],
    )(x, scale, beta2, root1, bias1.reshape(1, h))

    xs = _sc_gather(xq.reshape(n, f_in), src2d, f_in)

    epb = 4096
    nblk = e_total // epb
    qrows = epb // 4
    m1p = pl.pallas_call(
        functools.partial(_msg1_body, f_in=f_in),
        grid=(nblk,),
        in_specs=[pl.BlockSpec((dd, qrows), lambda j: (0, j)),
                  pl.BlockSpec((dd, qrows), lambda j: (0, j)),
                  pl.BlockSpec((dd, qrows), lambda j: (0, j)),
                  pl.BlockSpec((dd, qrows), lambda j: (0, j)),
                  pl.BlockSpec((qrows, 4 * f_in), lambda j: (j, 0)),
                  pl.BlockSpec((dd, dd * f_in), lambda j: (0, 0)),
                  pl.BlockSpec((f_in, dd * f_in), lambda j: (0, 0)),
                  pl.BlockSpec((dd * f_in, h), lambda j: (0, 0))],
        out_specs=pl.BlockSpec((epb // 2, 2 * h), lambda j: (j, 0)),
        out_shape=jax.ShapeDtypeStruct((e_total // 2, 2 * h), F32),
    )(ea_q[0], ea_q[1], ea_q[2], ea_q[3], xs.reshape(e_total // 4, 4 * f_in),
      s1, r1m, v1)

    agg1 = _sc_scatter_add(m1p.reshape(e_total, h), dst2d, n, h)

    h1p, r2p = pl.pallas_call(
        _hidden_body,
        out_shape=[jax.ShapeDtypeStruct((n // 2, 2 * h), F32),
                   jax.ShapeDtypeStruct((n // 2, 2 * h), F32)],
    )(agg1.reshape(2, n // 2, 2 * h), r1p, root2, bias2.reshape(1, h))

    h1s = _sc_gather(h1p.reshape(n, h), src2d, h)

    prows2 = epb // 2
    m2p = pl.pallas_call(
        functools.partial(_msg2_body, f_in=h),
        grid=(nblk,),
        in_specs=[pl.BlockSpec((dd, prows2), lambda j: (0, j)),
                  pl.BlockSpec((dd, prows2), lambda j: (0, j)),
                  pl.BlockSpec((prows2, 2 * h), lambda j: (j, 0)),
                  pl.BlockSpec((dd, dd * h), lambda j: (0, 0)),
                  pl.BlockSpec((h, dd * h), lambda j: (0, 0)),
                  pl.BlockSpec((dd * h, h), lambda j: (0, 0))],
        out_specs=pl.BlockSpec((prows2, 2 * h), lambda j: (j, 0)),
        out_shape=jax.ShapeDtypeStruct((e_total // 2, 2 * h), F32),
    )(ea_e, ea_o, h1s.reshape(e_total // 2, 2 * h), s2, r2m, v2)

    agg2 = _sc_scatter_add(m2p.reshape(e_total, h), dst2d, n, h)

    nb = 4
    bn = n // 2 // nb
    out = pl.pallas_call(
        functools.partial(_pool_body, n_graphs=N_GRAPHS, nblocks=nb, h=h),
        grid=(nb,),
        in_specs=[pl.BlockSpec((2, bn, 2 * h), lambda j: (0, j, 0)),
                  pl.BlockSpec((bn, 2 * h), lambda j: (j, 0)),
                  pl.BlockSpec((h, p_ch), lambda j: (0, 0)),
                  pl.BlockSpec((1, p_ch), lambda j: (0, 0)),
                  pl.BlockSpec((h, p_ch), lambda j: (0, 0)),
                  pl.BlockSpec((1, p_ch), lambda j: (0, 0)),
                  pl.BlockSpec((p_ch, n_out), lambda j: (0, 0)),
                  pl.BlockSpec((1, n_out), lambda j: (0, 0)),
                  pl.BlockSpec((1, bn), lambda j: (0, j)),
                  pl.BlockSpec((1, bn), lambda j: (0, j))],
        out_specs=pl.BlockSpec((N_GRAPHS, n_out), lambda j: (0, 0)),
        out_shape=jax.ShapeDtypeStruct((N_GRAPHS, n_out), F32),
        scratch_shapes=[pltpu.VMEM((N_GRAPHS, p_ch), F32)],
    )(agg2.reshape(2, n // 2, 2 * h), r2p, Wf, bf.reshape(1, p_ch),
      Wa, ba.reshape(1, p_ch), Wd, bd.reshape(1, n_out), seg_e, seg_o)
    return out


# speedup vs baseline: 1.3924x; 1.3924x over previous
import functools

import jax
import jax.numpy as jnp
from jax import lax
from jax.experimental import pallas as pl
from jax.experimental.pallas import tpu as pltpu
from jax.experimental.pallas import tpu_sc as plsc

F32 = jnp.float32
N_GRAPHS = 256
IDXBLK = 128
LANES = 128


def _pre_body(x_ref, scale_ref, beta_ref, root1_ref, bias1_ref,
              xq_ref, r1p_ref):
    n, f = x_ref.shape
    xn = x_ref[...] * scale_ref[...] + beta_ref[...]
    x4 = xn.reshape(n // 4, 4, f)
    xq_ref[...] = jnp.concatenate([x4[:, k, :] for k in range(4)], axis=1)
    r1 = jnp.dot(xn, root1_ref[...], preferred_element_type=F32) + bias1_ref[...]
    r2d = r1.reshape(n // 2, 2, r1.shape[1])
    r1p_ref[...] = jnp.concatenate([r2d[:, 0, :], r2d[:, 1, :]], axis=1)


def _msg1_body(ea_ref, xs_ref, s_ref, r_ref, v_ref, out_ref, *, f_in, dd):
    nq = xs_ref.shape[0]
    ea4 = ea_ref[...].T.reshape(nq, 4, dd)
    ms = []
    for k in range(4):
        xk = xs_ref[:, k * f_in:(k + 1) * f_in]
        e7 = jnp.dot(ea4[:, k, :], s_ref[...], preferred_element_type=F32)
        z = e7 * jnp.dot(xk, r_ref[...], preferred_element_type=F32)
        ms.append(jnp.dot(z, v_ref[...], preferred_element_type=F32))
    a = jnp.concatenate([ms[0], ms[1]], axis=1)
    b = jnp.concatenate([ms[2], ms[3]], axis=1)
    inter = jnp.concatenate([a[:, None, :], b[:, None, :]], axis=1)
    out_ref[...] = inter.reshape(out_ref.shape)


def _msg2_body(ea_ref, xs_ref, s_ref, r_ref, v_ref, out_ref, *, f_in, dd):
    nq = xs_ref.shape[0]
    ea2 = ea_ref[...].T.reshape(nq, 2, dd)
    ms = []
    for k in range(2):
        xk = xs_ref[:, k * f_in:(k + 1) * f_in]
        e7 = jnp.dot(ea2[:, k, :], s_ref[...], preferred_element_type=F32)
        z = e7 * jnp.dot(xk, r_ref[...], preferred_element_type=F32)
        ms.append(jnp.dot(z, v_ref[...], preferred_element_type=F32))
    out_ref[...] = jnp.concatenate(ms, axis=1)


def _hidden_body(agg_ref, r1p_ref, root2_ref, bias2_ref, h1p_ref, r2p_ref):
    h = root2_ref.shape[0]
    he = jnp.maximum(agg_ref[0][:, :h] + agg_ref[1][:, :h] + r1p_ref[:, :h], 0.0)
    ho = jnp.maximum(agg_ref[0][:, h:] + agg_ref[1][:, h:] + r1p_ref[:, h:], 0.0)
    h1p_ref[...] = jnp.concatenate([he, ho], axis=1)
    r2e = jnp.dot(he, root2_ref[...], preferred_element_type=F32) + bias2_ref[...]
    r2o = jnp.dot(ho, root2_ref[...], preferred_element_type=F32) + bias2_ref[...]
    r2p_ref[...] = jnp.concatenate([r2e, r2o], axis=1)


def _pool_body(agg_ref, r2p_ref, wf_ref, bf_ref, wa_ref, ba_ref, wd_ref, bd_ref,
               sege_ref, sego_ref, out_ref, acc_ref, *, n_graphs, nblocks, h):
    j = pl.program_id(0)
    part = None
    for k, seg_ref in enumerate((sege_ref, sego_ref)):
        hk = jnp.maximum(agg_ref[0][:, k * h:(k + 1) * h]
                         + agg_ref[1][:, k * h:(k + 1) * h]
                         + r2p_ref[:, k * h:(k + 1) * h], 0.0)
        feat = jnp.dot(hk, wf_ref[...], preferred_element_type=F32) + bf_ref[...]
        attn = jax.nn.sigmoid(jnp.dot(hk, wa_ref[...], preferred_element_type=F32)
                              + ba_ref[...])
        p = feat * attn
        seg = seg_ref[...]
        onehot = (seg == lax.broadcasted_iota(
            jnp.int32, (n_graphs, seg.shape[1]), 0)).astype(F32)
        pk = jnp.dot(onehot, p, preferred_element_type=F32)
        part = pk if part is None else part + pk

    @pl.when(j == 0)
    def _():
        acc_ref[...] = part

    @pl.when(j > 0)
    def _():
        acc_ref[...] = acc_ref[...] + part

    @pl.when(j == nblocks - 1)
    def _():
        out_ref[...] = (jnp.dot(acc_ref[...], wd_ref[...], preferred_element_type=F32)
                        + bd_ref[...])


_SC_PARAMS = pltpu.CompilerParams(use_tc_tiling_on_sc=False)


def _sc_gather(table, idx2d, fd):
    nrows_idx, _ = idx2d.shape
    e_total = nrows_idx * IDXBLK
    info = plsc.get_sparse_core_info()
    nc, ns = info.num_cores, info.num_subcores
    nw = nc * ns
    chunk = e_total // nw
    kblk = chunk // IDXBLK
    nparts = 4
    kpart = kblk // nparts
    prows = chunk // nparts
    nbuf = 2
    mesh = plsc.VectorSubcoreMesh(core_axis_name="c", subcore_axis_name="s")

    @functools.partial(
        pl.kernel,
        out_type=jax.ShapeDtypeStruct((e_total, fd), F32),
        mesh=mesh,
        compiler_params=_SC_PARAMS,
        scratch_types=[
            pltpu.VMEM((kblk, IDXBLK), jnp.int32),
            [pltpu.VMEM((prows, fd), F32) for _ in range(nbuf)],
            [pltpu.SemaphoreType.DMA for _ in range(nbuf)],
            [pltpu.SemaphoreType.DMA for _ in range(nbuf)],
        ],
    )
    def gk(table_hbm, idx_hbm, out_hbm, idx_v, bufs, gsems, osems):
        c = lax.axis_index("c")
        s = lax.axis_index("s")
        w = s * nc + c
        pltpu.sync_copy(idx_hbm.at[pl.ds(w * kblk, kblk)], idx_v)
        outcp = [None] * nbuf
        for p in range(nparts):
            b = p % nbuf
            if outcp[b] is not None:
                outcp[b].wait()
            cps = []
            for j in range(kpart):
                cps.append(pltpu.async_copy(
                    table_hbm.at[idx_v.at[p * kpart + j]],
                    bufs[b].at[pl.ds(j * IDXBLK, IDXBLK)], gsems[b]))
            for cp in cps:
                cp.wait()
            outcp[b] = pltpu.async_copy(
                bufs[b], out_hbm.at[pl.ds(w * chunk + p * prows, prows)], osems[b])
        for cp in outcp:
            cp.wait()

    return gk(table, idx2d)


def _sc_scatter_add(vals, idx2d, n_nodes, fd):
    nrows_idx, _ = idx2d.shape
    e_total = nrows_idx * IDXBLK
    info = plsc.get_sparse_core_info()
    nc, ns = info.num_cores, info.num_subcores
    nw = nc * ns
    chunk = e_total // nw
    kblk = chunk // IDXBLK
    nparts = kblk
    rows_per_tile = n_nodes // ns
    mesh = plsc.VectorSubcoreMesh(core_axis_name="c", subcore_axis_name="s")

    zrows = 16
    nbuf = 3

    @functools.partial(
        pl.kernel,
        out_type=jax.ShapeDtypeStruct((nc, n_nodes, fd), F32),
        mesh=mesh,
        compiler_params=_SC_PARAMS,
        scratch_types=[
            pltpu.VMEM((kblk, IDXBLK), jnp.int32),
            [pltpu.VMEM((IDXBLK, fd), F32) for _ in range(nbuf)],
            pltpu.VMEM((zrows, fd), F32),
            pltpu.VMEM_SHARED((n_nodes, fd), F32),
            [pltpu.SemaphoreType.DMA for _ in range(nbuf)],
            [pltpu.SemaphoreType.DMA for _ in range(nbuf)],
        ],
    )
    def sk(vals_hbm, idx_hbm, out_hbm, idx_v, bufs, zbuf, acc_sh, lsems, asems):
        c = lax.axis_index("c")
        s = lax.axis_index("s")
        w = s * nc + c
        r0 = s * rows_per_tile
        pltpu.sync_copy(idx_hbm.at[pl.ds(w * kblk, kblk)], idx_v)
        loadcp = [None] * nparts
        for p in range(nbuf):
            loadcp[p] = pltpu.async_copy(
                vals_hbm.at[pl.ds(w * chunk + p * IDXBLK, IDXBLK)],
                bufs[p], lsems[p])
        nlane16 = fd // 16

        def bz(k, _):
            zbuf[k // nlane16, pl.ds((k % nlane16) * 16, 16)] = jnp.zeros((16,), F32)
            return 0

        lax.fori_loop(0, zrows * nlane16, bz, 0)
        for t in range(rows_per_tile // zrows):
            pltpu.sync_copy(zbuf, acc_sh.at[pl.ds(r0 + t * zrows, zrows)])
        plsc.subcore_barrier()
        addcp = [None] * nparts
        for p in range(nparts):
            b = p % nbuf
            loadcp[p].wait()
            addcp[p] = pltpu.async_copy(bufs[b], acc_sh.at[idx_v.at[p]],
                                        asems[b], add=True)
            q = p + nbuf
            if q < nparts:
                addcp[p].wait()
                loadcp[q] = pltpu.async_copy(
                    vals_hbm.at[pl.ds(w * chunk + q * IDXBLK, IDXBLK)],
                    bufs[b], lsems[b])
        for p in range(nparts - nbuf, nparts):
            addcp[p].wait()
        plsc.subcore_barrier()
        pltpu.sync_copy(acc_sh.at[pl.ds(r0, rows_per_tile)],
                        out_hbm.at[c, pl.ds(r0, rows_per_tile)])

    return sk(vals, idx2d)


def kernel(x, e, gamma, beta, W1e, b1e, root1, bias1, W2e, b2e, root2, bias2,
           Wf, bf, Wa, ba, Wd, bd, edge_index, i):
    n, f_in = x.shape
    e_total, d_edge = e.shape
    h = root1.shape[1]
    p_ch = Wf.shape[1]
    n_out = Wd.shape[1]

    scale = (gamma * lax.rsqrt(jnp.float32(1.0 + 1e-3))).reshape(1, f_in)
    beta2 = beta.reshape(1, f_in)
    src2d = edge_index[0].reshape(e_total // IDXBLK, IDXBLK)
    dst2d = edge_index[1].reshape(e_total // IDXBLK, IDXBLK)
    dd = d_edge + 1
    ea_t = jnp.concatenate([e.T, jnp.ones((1, e_total), F32)], axis=0)
    v1 = jnp.concatenate([W1e, b1e[None, :]], axis=0).reshape(dd * f_in, h)
    v2 = jnp.concatenate([W2e, b2e[None, :]], axis=0).reshape(dd * h, h)
    s1 = jnp.kron(jnp.eye(dd, dtype=F32), jnp.ones((1, f_in), F32))
    r1m = jnp.tile(jnp.eye(f_in, dtype=F32), (1, dd))
    s2 = jnp.kron(jnp.eye(dd, dtype=F32), jnp.ones((1, h), F32))
    r2m = jnp.tile(jnp.eye(h, dtype=F32), (1, dd))
    seg_e = i[0::2].reshape(1, n // 2)
    seg_o = i[1::2].reshape(1, n // 2)

    xq, r1p = pl.pallas_call(
        _pre_body,
        out_shape=[jax.ShapeDtypeStruct((n // 4, 4 * f_in), F32),
                   jax.ShapeDtypeStruct((n // 2, 2 * h), F32)],
    )(x, scale, beta2, root1, bias1.reshape(1, h))

    xs = _sc_gather(xq.reshape(n, f_in), src2d, f_in)

    epb = 4096
    nblk = e_total // epb
    qrows = epb // 4
    m1p = pl.pallas_call(
        functools.partial(_msg1_body, f_in=f_in, dd=dd),
        grid=(nblk,),
        in_specs=[pl.BlockSpec((dd, epb), lambda j: (0, j)),
                  pl.BlockSpec((qrows, 4 * f_in), lambda j: (j, 0)),
                  pl.BlockSpec((dd, dd * f_in), lambda j: (0, 0)),
                  pl.BlockSpec((f_in, dd * f_in), lambda j: (0, 0)),
                  pl.BlockSpec((dd * f_in, h), lambda j: (0, 0))],
        out_specs=pl.BlockSpec((epb // 2, 2 * h), lambda j: (j, 0)),
        out_shape=jax.ShapeDtypeStruct((e_total // 2, 2 * h), F32),
    )(ea_t, xs.reshape(e_total // 4, 4 * f_in), s1, r1m, v1)

    agg1 = _sc_scatter_add(m1p.reshape(e_total, h), dst2d, n, h)

    h1p, r2p = pl.pallas_call(
        _hidden_body,
        out_shape=[jax.ShapeDtypeStruct((n // 2, 2 * h), F32),
                   jax.ShapeDtypeStruct((n // 2, 2 * h), F32)],
    )(agg1.reshape(2, n // 2, 2 * h), r1p, root2, bias2.reshape(1, h))

    h1s = _sc_gather(h1p.reshape(n, h), src2d, h)

    prows2 = epb // 2
    m2p = pl.pallas_call(
        functools.partial(_msg2_body, f_in=h, dd=dd),
        grid=(nblk,),
        in_specs=[pl.BlockSpec((dd, epb), lambda j: (0, j)),
                  pl.BlockSpec((prows2, 2 * h), lambda j: (j, 0)),
                  pl.BlockSpec((dd, dd * h), lambda j: (0, 0)),
                  pl.BlockSpec((h, dd * h), lambda j: (0, 0)),
                  pl.BlockSpec((dd * h, h), lambda j: (0, 0))],
        out_specs=pl.BlockSpec((prows2, 2 * h), lambda j: (j, 0)),
        out_shape=jax.ShapeDtypeStruct((e_total // 2, 2 * h), F32),
    )(ea_t, h1s.reshape(e_total // 2, 2 * h), s2, r2m, v2)

    agg2 = _sc_scatter_add(m2p.reshape(e_total, h), dst2d, n, h)

    nb = 4
    bn = n // 2 // nb
    out = pl.pallas_call(
        functools.partial(_pool_body, n_graphs=N_GRAPHS, nblocks=nb, h=h),
        grid=(nb,),
        in_specs=[pl.BlockSpec((2, bn, 2 * h), lambda j: (0, j, 0)),
                  pl.BlockSpec((bn, 2 * h), lambda j: (j, 0)),
                  pl.BlockSpec((h, p_ch), lambda j: (0, 0)),
                  pl.BlockSpec((1, p_ch), lambda j: (0, 0)),
                  pl.BlockSpec((h, p_ch), lambda j: (0, 0)),
                  pl.BlockSpec((1, p_ch), lambda j: (0, 0)),
                  pl.BlockSpec((p_ch, n_out), lambda j: (0, 0)),
                  pl.BlockSpec((1, n_out), lambda j: (0, 0)),
                  pl.BlockSpec((1, bn), lambda j: (0, j)),
                  pl.BlockSpec((1, bn), lambda j: (0, j))],
        out_specs=pl.BlockSpec((N_GRAPHS, n_out), lambda j: (0, 0)),
        out_shape=jax.ShapeDtypeStruct((N_GRAPHS, n_out), F32),
        scratch_shapes=[pltpu.VMEM((N_GRAPHS, p_ch), F32)],
    )(agg2.reshape(2, n // 2, 2 * h), r2p, Wf, bf.reshape(1, p_ch),
      Wa, ba.reshape(1, p_ch), Wd, bd.reshape(1, n_out), seg_e, seg_o)
    return out
